# baseline (speedup 1.0000x reference)
import jax
import jax.numpy as jnp
from jax import lax
from jax.experimental import pallas as pl
from jax.experimental.pallas import tpu as pltpu
from jax.experimental.pallas import tpu_sc as plsc

BATCH = 16384
EMBED = 16
NC = 2
NS = 16
NW = NC * NS
NT = 7813
TPW = 245
WT = 8
WCOLS = WT * 128
NWIN = 31
LCAP = 2048
BCAP = 64
NVREG = BATCH // 16


def _ncf_body(uidx_hbm, iidx_hbm, utab_t, itab_t, wb_hbm, p0_hbm, p1_hbm,
              idxbuf, lidx, lpos, bidx, bpos, bvals, winbuf, stage, wb_v,
              shared, sems):
    cid = lax.axis_index("c")
    sid = lax.axis_index("s")
    wid = sid * NC + cid
    t0 = wid * TPW
    t1 = jnp.minimum(t0 + TPW, NT)

    pltpu.sync_copy(wb_hbm, wb_v)
    wu_vec = wb_v[pl.ds(0, 16)]
    wv_vec = wb_v[pl.ds(16, 16)]
    b_vec = wb_v[pl.ds(32, 16)]
    iota = lax.iota(jnp.int32, 16)
    zeros_f = jnp.zeros((16,), jnp.float32)
    zeros_i = jnp.zeros((16,), jnp.int32)
    neg1 = zeros_i - 1

    seed = jnp.where(cid == 0, b_vec[0], 0.0)
    seed_vec = zeros_f + seed
    for k in range(64):
        stage[pl.ds(k * 16, 16)] = seed_vec
    pltpu.sync_copy(stage, shared.at[pl.ds(sid * 1024, 1024)])
    plsc.subcore_barrier()

    def fetch(tab, win, buf):
        cb = jnp.minimum(t0 + win * WT, NT - WT) * 128
        cb = pl.multiple_of(cb, 128)
        pltpu.async_copy(tab.at[:, pl.ds(cb, WCOLS)], winbuf.at[buf],
                         sems.at[buf])

    def drain(tab, buf):
        pltpu.make_async_copy(tab.at[:, pl.ds(0, WCOLS)], winbuf.at[buf],
                              sems.at[buf]).wait()

    def table_pass(tab, idx_hbm, w_vec):
        pltpu.sync_copy(idx_hbm, idxbuf)

        for k in range(LCAP // 16):
            lidx[pl.ds(k * 16, 16)] = neg1
        for w in range(NWIN):
            for g in range(8):
                bidx[w, pl.ds(g * 16, 16)] = neg1
                bpos[w, pl.ds(g * 16, 16)] = zeros_i
                bvals[w, pl.ds(g * 16, 16)] = zeros_f

        def scan_blk(blk, off):
            for u in range(4):
                k = blk * 4 + u
                vec = idxbuf[pl.ds(k * 16, 16)]
                tile = lax.shift_right_logical(vec, 7)
                m = jnp.logical_and(tile >= t0, tile < t1)
                cum = plsc.cumsum(m.astype(jnp.int32))
                tgt = jnp.minimum(off + cum - 1, LCAP - 1)
                plsc.store_scatter(lidx, [tgt], vec, mask=m)
                plsc.store_scatter(lpos, [tgt], k * 16 + iota, mask=m)
                off = off + cum[15]
            return off

        cnt = lax.fori_loop(0, NVREG // 4, scan_blk, 0)
        nv = jnp.minimum(lax.shift_right_logical(cnt + 15, 4), LCAP // 16)

        def bin_win(win, carry):
            def bin_vreg(k, woff):
                lv = lidx[pl.ds(k * 16, 16)]
                pv = lpos[pl.ds(k * 16, 16)]
                tile = lax.shift_right_logical(lv, 7)
                winv = lax.shift_right_logical(tile - t0, 3)
                m = jnp.logical_and(lv >= 0, winv == win)
                cum = plsc.cumsum(m.astype(jnp.int32))
                tgt = jnp.minimum(woff + cum - 1, BCAP - 1)
                wsplat = zeros_i + win
                plsc.store_scatter(bidx, [wsplat, tgt], lv, mask=m)
                plsc.store_scatter(bpos, [wsplat, tgt], pv, mask=m)
                return woff + cum[15]

            lax.fori_loop(0, nv, bin_vreg, 0)
            return carry

        lax.fori_loop(0, NWIN, bin_win, 0)

        fetch(tab, 0, 0)
        fetch(tab, 1, 1)

        def win_body(win, carry):
            buf = lax.rem(win, 3)

            @pl.when(win + 2 < NWIN)
            def _():
                fetch(tab, win + 2, lax.rem(win + 2, 3))

            drain(tab, buf)
            cb = jnp.minimum(t0 + win * WT, NT - WT) * 128
            buf_splat = zeros_i + buf
            for g in range(BCAP // 16):
                iv = bidx[win, pl.ds(g * 16, 16)]
                m = iv >= 0
                local = lax.bitwise_and(iv - cb, WCOLS - 1)
                acc = zeros_f
                for e in range(EMBED):
                    esplat = jnp.full((16,), e, jnp.int32)
                    vals = plsc.load_gather(winbuf, [buf_splat, esplat, local])
                    acc = acc + vals * w_vec[e]
                bvals[win, pl.ds(g * 16, 16)] = jnp.where(m, acc, 0.0)
            return carry

        lax.fori_loop(0, NWIN, win_body, 0)

        for w in range(NWIN):
            pltpu.sync_copy(bvals.at[w], shared.at[bpos.at[w]], add=True)

    table_pass(utab_t, uidx_hbm, wu_vec)
    table_pass(itab_t, iidx_hbm, wv_vec)

    plsc.subcore_barrier()

    @pl.when(cid == 0)
    def _():
        pltpu.sync_copy(shared.at[pl.ds(sid * 1024, 1024)],
                        p0_hbm.at[pl.ds(sid * 1024, 1024)])

    @pl.when(cid == 1)
    def _():
        pltpu.sync_copy(shared.at[pl.ds(sid * 1024, 1024)],
                        p1_hbm.at[pl.ds(sid * 1024, 1024)])


def _combine_body(p0_ref, p1_ref, o_ref):
    o_ref[...] = p0_ref[...] + p1_ref[...]


@jax.jit
def _ncf(uidx, iidx, utab_t, itab_t, wb):
    mesh = plsc.VectorSubcoreMesh(core_axis_name="c", subcore_axis_name="s")
    kern = pl.kernel(
        _ncf_body,
        mesh=mesh,
        compiler_params=pltpu.CompilerParams(
            needs_layout_passes=False, use_tc_tiling_on_sc=True),
        out_type=(jax.ShapeDtypeStruct((BATCH,), jnp.float32),
                  jax.ShapeDtypeStruct((BATCH,), jnp.float32)),
        scratch_types=[
            pltpu.VMEM((BATCH,), jnp.int32),
            pltpu.VMEM((LCAP,), jnp.int32),
            pltpu.VMEM((LCAP,), jnp.int32),
            pltpu.VMEM((NWIN, 128), jnp.int32),
            pltpu.VMEM((NWIN, 128), jnp.int32),
            pltpu.VMEM((NWIN, 128), jnp.float32),
            pltpu.VMEM((3, EMBED, WCOLS), jnp.float32),
            pltpu.VMEM((1024,), jnp.float32),
            pltpu.VMEM((48,), jnp.float32),
            pltpu.VMEM_SHARED((BATCH,), jnp.float32),
            pltpu.SemaphoreType.DMA((3,)),
        ],
    )
    p0, p1 = kern(uidx, iidx, utab_t, itab_t, wb)
    return pl.pallas_call(
        _combine_body,
        out_shape=jax.ShapeDtypeStruct((BATCH,), jnp.float32),
    )(p0, p1)


def kernel(user_indices, item_indices, user_table, item_table, W, b):
    uidx = user_indices.astype(jnp.int32)
    iidx = item_indices.astype(jnp.int32)
    utab_t = user_table.T
    itab_t = item_table.T
    wb = jnp.concatenate([W[:, 0], b, jnp.zeros((15,), jnp.float32)])
    return _ncf(uidx, iidx, utab_t, itab_t, wb)


# speedup vs baseline: 2.0831x; 2.0831x over previous
import jax
import jax.numpy as jnp
from jax import lax
from jax.experimental import pallas as pl
from jax.experimental.pallas import tpu as pltpu
from jax.experimental.pallas import tpu_sc as plsc

BATCH = 16384
EMBED = 16
NC = 2
NS = 16
NW = NC * NS
BPW = BATCH // NW
C = 16
NCH = BPW // C
NBUF = 3


def _ncf_body(uidx_hbm, iidx_hbm, utab_t, itab_t, wb_hbm, out_hbm,
              uidx_v, iidx_v, tbuf, out_v, wb_v, sems):
    wid = lax.axis_index("s") * NC + lax.axis_index("c")
    base = wid * BPW

    pltpu.sync_copy(uidx_hbm.at[pl.ds(base, BPW)], uidx_v)
    pltpu.sync_copy(iidx_hbm.at[pl.ds(base, BPW)], iidx_v)
    pltpu.sync_copy(wb_hbm, wb_v)

    wu_vec = wb_v[pl.ds(0, 16)]
    wv_vec = wb_v[pl.ds(16, 16)]
    b_vec = wb_v[pl.ds(32, 16)]
    bias = jnp.zeros((16,), jnp.float32) + b_vec[0]
    iota = lax.iota(jnp.int32, 16)
    row_base = iota * 16

    def fetch(tab, idx_v, ch, buf):
        tvec = lax.shift_right_logical(idx_v[pl.ds(ch * C, C)], 7)
        for j in range(C):
            col0 = pl.multiple_of(tvec[j] * 128, 128)
            pltpu.async_copy(tab.at[:, pl.ds(col0, 128)],
                             tbuf.at[buf, pl.ds(j * 16, 16), :],
                             sems.at[buf])

    def drain(buf):
        for _ in range(C):
            pltpu.make_async_copy(utab_t.at[:, pl.ds(0, 128)],
                                  tbuf.at[buf, pl.ds(0, 16), :],
                                  sems.at[buf]).wait()

    def table_pass(tab, idx_v, w_vec, first):
        fetch(tab, idx_v, 0, 0)
        fetch(tab, idx_v, 1, 1)

        def chunk_body(ch, carry):
            buf = lax.rem(ch, NBUF)
            nxt = lax.rem(ch + 2, NBUF)

            @pl.when(ch + 2 < NCH)
            def _():
                fetch(tab, idx_v, ch + 2, nxt)

            drain(buf)

            lvec = lax.bitwise_and(idx_v[pl.ds(ch * C, C)], 127)
            buf_splat = jnp.zeros((16,), jnp.int32) + buf
            if first:
                acc = bias
            else:
                acc = out_v[pl.ds(ch * C, C)]
            for e in range(EMBED):
                rows = row_base + e
                vals = plsc.load_gather(tbuf, [buf_splat, rows, lvec])
                acc = acc + vals * w_vec[e]
            out_v[pl.ds(ch * C, C)] = acc
            return carry

        lax.fori_loop(0, NCH, chunk_body, 0)

    table_pass(utab_t, uidx_v, wu_vec, True)
    table_pass(itab_t, iidx_v, wv_vec, False)

    pltpu.sync_copy(out_v, out_hbm.at[pl.ds(base, BPW)])


@jax.jit
def _ncf(uidx, iidx, utab_t, itab_t, wb):
    mesh = plsc.VectorSubcoreMesh(core_axis_name="c", subcore_axis_name="s")
    kern = pl.kernel(
        _ncf_body,
        mesh=mesh,
        compiler_params=pltpu.CompilerParams(
            needs_layout_passes=False, use_tc_tiling_on_sc=True),
        out_type=jax.ShapeDtypeStruct((BATCH,), jnp.float32),
        scratch_types=[
            pltpu.VMEM((BPW,), jnp.int32),
            pltpu.VMEM((BPW,), jnp.int32),
            pltpu.VMEM((NBUF, C * 16, 128), jnp.float32),
            pltpu.VMEM((BPW,), jnp.float32),
            pltpu.VMEM((48,), jnp.float32),
            pltpu.SemaphoreType.DMA((NBUF,)),
        ],
    )
    return kern(uidx, iidx, utab_t, itab_t, wb)


def kernel(user_indices, item_indices, user_table, item_table, W, b):
    uidx = user_indices.astype(jnp.int32)
    iidx = item_indices.astype(jnp.int32)
    utab_t = user_table.T
    itab_t = item_table.T
    wb = jnp.concatenate([W[:, 0], b, jnp.zeros((15,), jnp.float32)])
    return _ncf(uidx, iidx, utab_t, itab_t, wb)


# speedup vs baseline: 2.1305x; 1.0228x over previous
import jax
import jax.numpy as jnp
from jax import lax
from jax.experimental import pallas as pl
from jax.experimental.pallas import tpu as pltpu
from jax.experimental.pallas import tpu_sc as plsc

BATCH = 16384
EMBED = 16
NC = 2
NS = 16
NW = NC * NS
BPW = BATCH // NW
C = 16
NCH = BPW // C
NBUF = 3


def _ncf_body(uidx_hbm, iidx_hbm, utab_t, itab_t, wb_hbm, out_hbm,
              uidx_v, iidx_v, tbuf, out_v, wb_v, sems):
    wid = lax.axis_index("s") * NC + lax.axis_index("c")
    base = wid * BPW

    pltpu.sync_copy(uidx_hbm.at[pl.ds(base, BPW)], uidx_v)
    pltpu.sync_copy(iidx_hbm.at[pl.ds(base, BPW)], iidx_v)
    pltpu.sync_copy(wb_hbm, wb_v)

    wu_vec = wb_v[pl.ds(0, 16)]
    wv_vec = wb_v[pl.ds(16, 16)]
    b_vec = wb_v[pl.ds(32, 16)]
    bias = jnp.zeros((16,), jnp.float32) + b_vec[0]
    iota = lax.iota(jnp.int32, 16)
    row_base = iota * 16

    def fetch(tab, idx_v, ch, buf):
        tvec = lax.shift_right_logical(idx_v[pl.ds(ch * C, C)], 7)
        for j in range(C):
            col0 = pl.multiple_of(tvec[j] * 128, 128)
            pltpu.async_copy(tab.at[:, pl.ds(col0, 128)],
                             tbuf.at[buf, pl.ds(j * 16, 16), :],
                             sems.at[buf])

    def drain(buf):
        for _ in range(C):
            pltpu.make_async_copy(utab_t.at[:, pl.ds(0, 128)],
                                  tbuf.at[buf, pl.ds(0, 16), :],
                                  sems.at[buf]).wait()

    def fetch_slot(s, buf):
        ch = lax.shift_right_logical(s, 1)

        @pl.when(lax.rem(s, 2) == 0)
        def _():
            fetch(utab_t, uidx_v, ch, buf)

        @pl.when(lax.rem(s, 2) == 1)
        def _():
            fetch(itab_t, iidx_v, ch, buf)

    fetch_slot(0, 0)
    fetch_slot(1, 1)

    def slot_body(s, carry):
        buf = lax.rem(s, NBUF)

        @pl.when(s + 2 < 2 * NCH)
        def _():
            fetch_slot(s + 2, lax.rem(s + 2, NBUF))

        drain(buf)

        ch = lax.shift_right_logical(s, 1)
        is_user = lax.rem(s, 2) == 0
        buf_splat = jnp.zeros((16,), jnp.int32) + buf

        @pl.when(is_user)
        def _():
            lvec = lax.bitwise_and(uidx_v[pl.ds(ch * C, C)], 127)
            acc = bias
            for e in range(EMBED):
                vals = plsc.load_gather(tbuf, [buf_splat, row_base + e, lvec])
                acc = acc + vals * wu_vec[e]
            out_v[pl.ds(ch * C, C)] = acc

        @pl.when(jnp.logical_not(is_user))
        def _():
            lvec = lax.bitwise_and(iidx_v[pl.ds(ch * C, C)], 127)
            acc = out_v[pl.ds(ch * C, C)]
            for e in range(EMBED):
                vals = plsc.load_gather(tbuf, [buf_splat, row_base + e, lvec])
                acc = acc + vals * wv_vec[e]
            out_v[pl.ds(ch * C, C)] = acc

        return carry

    lax.fori_loop(0, 2 * NCH, slot_body, 0)

    pltpu.sync_copy(out_v, out_hbm.at[pl.ds(base, BPW)])


@jax.jit
def _ncf(uidx, iidx, utab_t, itab_t, wb):
    mesh = plsc.VectorSubcoreMesh(core_axis_name="c", subcore_axis_name="s")
    kern = pl.kernel(
        _ncf_body,
        mesh=mesh,
        compiler_params=pltpu.CompilerParams(
            needs_layout_passes=False, use_tc_tiling_on_sc=True),
        out_type=jax.ShapeDtypeStruct((BATCH,), jnp.float32),
        scratch_types=[
            pltpu.VMEM((BPW,), jnp.int32),
            pltpu.VMEM((BPW,), jnp.int32),
            pltpu.VMEM((NBUF, C * 16, 128), jnp.float32),
            pltpu.VMEM((BPW,), jnp.float32),
            pltpu.VMEM((48,), jnp.float32),
            pltpu.SemaphoreType.DMA((NBUF,)),
        ],
    )
    return kern(uidx, iidx, utab_t, itab_t, wb)


def kernel(user_indices, item_indices, user_table, item_table, W, b):
    uidx = user_indices.astype(jnp.int32)
    iidx = item_indices.astype(jnp.int32)
    utab_t = user_table.T
    itab_t = item_table.T
    wb = jnp.concatenate([W[:, 0], b, jnp.zeros((15,), jnp.float32)])
    return _ncf(uidx, iidx, utab_t, itab_t, wb)
